# baseline (speedup 1.0000x reference)
import functools

import jax
import jax.numpy as jnp
from jax import lax
from jax.experimental import pallas as pl
from jax.experimental.pallas import tpu as pltpu
from jax.experimental.pallas import tpu_sc as plsc

N = 10000
E = 320000
D_IN = 128
D_HID = 128
D_OUT = 64

NC = 2
NS = 16
NW = NC * NS
NPAD = 10240
PER_SUB = NPAD // NS
E_TILE = E // NW
K = 128
CHUNKS = 80
E_SUB = E // NS
CHUNKS_F = 160
DHALF = D_HID // 2
NBUF = 2
DST_PAD = N + 8

_LANES = 16


_MESH = plsc.VectorSubcoreMesh(
    core_axis_name="c", subcore_axis_name="s", num_cores=NC, num_subcores=NS
)


def _mesh():
    return _MESH


@functools.partial(
    pl.kernel,
    out_type=jax.ShapeDtypeStruct((NC * NPAD,), jnp.float32),
    mesh=_mesh(),
    scratch_types=[
        pltpu.VMEM((CHUNKS + 1, K), jnp.int32),
        pltpu.VMEM((K,), jnp.float32),
        pltpu.VMEM_SHARED((NPAD,), jnp.float32),
    ]
    + [pltpu.SemaphoreType.DMA] * 4,
)
def _deg_kernel(dstr_hbm, out_hbm, didx, ones, acc, *sems):
    c = lax.axis_index("c")
    s = lax.axis_index("s")
    wid = c * NS + s

    pltpu.sync_copy(dstr_hbm.at[wid], didx)

    def fill(i, _):
        ones[pl.ds(i * _LANES, _LANES)] = jnp.ones((_LANES,), jnp.float32)
        return _

    lax.fori_loop(0, K // _LANES, fill, None)

    def init(j, _):
        pltpu.sync_copy(ones, acc.at[pl.ds(s * PER_SUB + j * K, K)])
        return _

    lax.fori_loop(0, PER_SUB // K, init, None)
    plsc.subcore_barrier()

    def outer(o, _):
        for b in range(4):
            i = o * 4 + b
            pltpu.async_copy(ones, acc.at[didx.at[i]], sems[b], add=True)
        for b in range(4):
            pltpu.make_async_copy(ones, acc.at[didx.at[0]], sems[b]).wait()
        return _

    lax.fori_loop(0, CHUNKS // 4, outer, None)
    plsc.subcore_barrier()

    def out(j, _):
        off = s * PER_SUB + j * K
        pltpu.sync_copy(acc.at[pl.ds(off, K)], ones)
        pltpu.sync_copy(ones, out_hbm.at[pl.ds(c * NPAD + off, K)])
        return _

    lax.fori_loop(0, PER_SUB // K, out, None)


KE = 40
ECHUNKS = E_TILE // KE


def _make_scatter():
    @functools.partial(
        pl.kernel,
        out_type=jax.ShapeDtypeStruct((NC, NPAD, D_HID), jnp.float32),
        mesh=_mesh(),
        scratch_types=[
            pltpu.VMEM((KE,), jnp.int32),
            pltpu.VMEM((KE,), jnp.int32),
            pltpu.VMEM((KE, D_HID), jnp.float32),
            pltpu.VMEM_SHARED((NPAD, D_HID), jnp.float32),
            pltpu.SemaphoreType.DMA,
        ],
    )
    def scatter_kernel(y_hbm, src_hbm, dst_hbm, out_hbm, sidx, didx, rows, acc, sem):
        c = lax.axis_index("c")
        s = lax.axis_index("s")
        wid = c * NS + s

        def zero(i, _):
            r = i // (D_HID // _LANES)
            col = (i % (D_HID // _LANES)) * _LANES
            rows[r, pl.ds(col, _LANES)] = jnp.zeros((_LANES,), jnp.float32)
            return _

        lax.fori_loop(0, KE * D_HID // _LANES, zero, None)

        def init(j, _):
            pltpu.sync_copy(rows, acc.at[pl.ds(s * PER_SUB + j * KE, KE)])
            return _

        lax.fori_loop(0, PER_SUB // KE, init, None)
        plsc.subcore_barrier()

        def body(i, _):
            base = pl.multiple_of(wid * E_TILE + i * KE, 8)
            pltpu.sync_copy(src_hbm.at[pl.ds(base, KE)], sidx)
            pltpu.sync_copy(dst_hbm.at[pl.ds(base, KE)], didx)
            pltpu.async_copy(y_hbm.at[sidx], rows, sem).wait()
            pltpu.sync_copy(rows, acc.at[didx], add=True)
            return _

        lax.fori_loop(0, ECHUNKS, body, None)
        plsc.subcore_barrier()

        def out(j, _):
            off = s * PER_SUB + j * KE
            pltpu.sync_copy(acc.at[pl.ds(off, KE)], rows)
            pltpu.sync_copy(rows, out_hbm.at[c, pl.ds(off, KE)])
            return _

        lax.fori_loop(0, PER_SUB // KE, out, None)

    return scatter_kernel


_scatter = _make_scatter()


_R = 2000


def _dinv_of(degp):
    return lax.rsqrt(degp[0] + degp[1] - 1.0)


def _t1_body(x_ref, w_ref, degp_ref, y_ref):
    dinv = _dinv_of(degp_ref[...])
    y_ref[...] = dinv * jnp.dot(
        x_ref[...], w_ref[...], preferred_element_type=jnp.float32
    )


def _t2_body(s_ref, y1_ref, degp_ref, b1_ref, w2_ref, y2_ref):
    dinv = _dinv_of(degp_ref[...])
    sblk = s_ref[...]
    h = jnp.maximum(dinv * (sblk[0] + sblk[1] + y1_ref[...]) + b1_ref[...], 0.0)
    y2_ref[...] = dinv * jnp.dot(h, w2_ref[...], preferred_element_type=jnp.float32)


def _t3_body(s_ref, y2_ref, degp_ref, b2_ref, o_ref):
    dinv = _dinv_of(degp_ref[...])
    sblk = s_ref[...]
    tot = (sblk[0] + sblk[1] + y2_ref[...])[:, :D_OUT]
    o_ref[...] = jnp.maximum(dinv * tot + b2_ref[...], 0.0)


def _row_spec(d):
    return pl.BlockSpec((_R, d), lambda i: (i, 0))


def _slab_spec(d):
    return pl.BlockSpec((NC, _R, d), lambda i: (0, i, 0))


def _full_spec(a, b):
    return pl.BlockSpec((a, b), lambda i: (0, 0))


_t1 = pl.pallas_call(
    _t1_body,
    grid=(N // _R,),
    in_specs=[_row_spec(D_IN), _full_spec(D_IN, D_HID), _slab_spec(1)],
    out_specs=_row_spec(D_HID),
    out_shape=jax.ShapeDtypeStruct((N, D_HID), jnp.float32),
)

_t2 = pl.pallas_call(
    _t2_body,
    grid=(N // _R,),
    in_specs=[
        _slab_spec(D_HID),
        _row_spec(D_HID),
        _slab_spec(1),
        _full_spec(1, D_HID),
        _full_spec(D_HID, D_HID),
    ],
    out_specs=_row_spec(D_HID),
    out_shape=jax.ShapeDtypeStruct((N, D_HID), jnp.float32),
)

_t3 = pl.pallas_call(
    _t3_body,
    grid=(N // _R,),
    in_specs=[_slab_spec(D_HID), _row_spec(D_HID), _slab_spec(1), _full_spec(1, D_OUT)],
    out_specs=_row_spec(D_OUT),
    out_shape=jax.ShapeDtypeStruct((N, D_OUT), jnp.float32),
)


_K_REAL = E // (NW * CHUNKS)


def _chunk32(v, fill):
    v3 = v.reshape(NW, CHUNKS, _K_REAL)
    return jnp.pad(
        v3, ((0, 0), (0, 1), (0, K - _K_REAL)), constant_values=fill
    )


def _chunk16(v, fill):
    v2 = jnp.pad(
        v.reshape(NS, E_SUB), ((0, 0), (0, CHUNKS_F * K - E_SUB)),
        constant_values=fill,
    )
    return v2.reshape(NS, CHUNKS_F, K)


def kernel(x, edge_index, W1, b1, W2, b2):
    ei = edge_index.astype(jnp.int32)
    src, dst = ei[0], ei[1]
    dstr = _chunk32(dst, DST_PAD)

    degp = _deg_kernel(dstr).reshape(NC, NPAD)
    degp3 = degp[:, :, None]

    W2p = jnp.pad(W2, ((0, 0), (0, D_HID - D_OUT)))

    y1 = _t1(x, W1, degp3)
    s1 = _scatter(y1, src, dst)
    y2 = _t2(s1, y1, degp3, b1[None, :], W2p)
    s2 = _scatter(y2, src, dst)
    return _t3(s2, y2, degp3, b2[None, :])


# speedup vs baseline: 1.5202x; 1.5202x over previous
import functools

import jax
import jax.numpy as jnp
from jax import lax
from jax.experimental import pallas as pl
from jax.experimental.pallas import tpu as pltpu
from jax.experimental.pallas import tpu_sc as plsc

N = 10000
E = 320000
D_IN = 128
D_HID = 128
D_OUT = 64

NC = 2
NS = 16
NW = NC * NS
NPAD = 10240
PER_SUB = NPAD // NS
E_TILE = E // NW
K = 128
CHUNKS = 80
DST_PAD = N + 8

_LANES = 16


_MESH = plsc.VectorSubcoreMesh(
    core_axis_name="c", subcore_axis_name="s", num_cores=NC, num_subcores=NS
)


def _mesh():
    return _MESH


@functools.partial(
    pl.kernel,
    out_type=jax.ShapeDtypeStruct((NC * NPAD,), jnp.float32),
    mesh=_mesh(),
    scratch_types=[
        pltpu.VMEM((CHUNKS + 1, K), jnp.int32),
        pltpu.VMEM((K,), jnp.float32),
        pltpu.VMEM_SHARED((NPAD,), jnp.float32),
    ]
    + [pltpu.SemaphoreType.DMA] * 4,
)
def _deg_kernel(dstr_hbm, out_hbm, didx, ones, acc, *sems):
    c = lax.axis_index("c")
    s = lax.axis_index("s")
    wid = c * NS + s

    pltpu.sync_copy(dstr_hbm.at[wid], didx)

    def fill(i, _):
        ones[pl.ds(i * _LANES, _LANES)] = jnp.ones((_LANES,), jnp.float32)
        return _

    lax.fori_loop(0, K // _LANES, fill, None)

    def init(j, _):
        pltpu.sync_copy(ones, acc.at[pl.ds(s * PER_SUB + j * K, K)])
        return _

    lax.fori_loop(0, PER_SUB // K, init, None)
    plsc.subcore_barrier()

    def outer(o, _):
        for b in range(4):
            i = o * 4 + b
            pltpu.async_copy(ones, acc.at[didx.at[i]], sems[b], add=True)
        for b in range(4):
            pltpu.make_async_copy(ones, acc.at[didx.at[0]], sems[b]).wait()
        return _

    lax.fori_loop(0, CHUNKS // 4, outer, None)
    plsc.subcore_barrier()

    def out(j, _):
        off = s * PER_SUB + j * K
        pltpu.sync_copy(acc.at[pl.ds(off, K)], ones)
        pltpu.sync_copy(ones, out_hbm.at[pl.ds(c * NPAD + off, K)])
        return _

    lax.fori_loop(0, PER_SUB // K, out, None)


KE = 80
ECHUNKS = E_TILE // KE


def _make_scatter():
    @functools.partial(
        pl.kernel,
        out_type=jax.ShapeDtypeStruct((NC, NPAD, D_HID), jnp.float32),
        mesh=_mesh(),
        scratch_types=[
            pltpu.VMEM((KE,), jnp.int32),
            pltpu.VMEM((KE,), jnp.int32),
            pltpu.VMEM((KE, D_HID), jnp.float32),
            pltpu.VMEM_SHARED((NPAD, D_HID), jnp.float32),
            pltpu.SemaphoreType.DMA,
        ],
    )
    def scatter_kernel(y_hbm, src_hbm, dst_hbm, out_hbm, sidx, didx, rows, acc, sem):
        c = lax.axis_index("c")
        s = lax.axis_index("s")
        wid = c * NS + s

        def zero(i, _):
            r = i // (D_HID // _LANES)
            col = (i % (D_HID // _LANES)) * _LANES
            rows[r, pl.ds(col, _LANES)] = jnp.zeros((_LANES,), jnp.float32)
            return _

        lax.fori_loop(0, KE * D_HID // _LANES, zero, None)

        def init(j, _):
            pltpu.sync_copy(rows, acc.at[pl.ds(s * PER_SUB + j * KE, KE)])
            return _

        lax.fori_loop(0, PER_SUB // KE, init, None)
        plsc.subcore_barrier()

        def body(i, _):
            base = pl.multiple_of(wid * E_TILE + i * KE, 8)
            pltpu.sync_copy(src_hbm.at[pl.ds(base, KE)], sidx)
            pltpu.sync_copy(dst_hbm.at[pl.ds(base, KE)], didx)
            pltpu.async_copy(y_hbm.at[sidx], rows, sem).wait()
            pltpu.sync_copy(rows, acc.at[didx], add=True)
            return _

        lax.fori_loop(0, ECHUNKS, body, None)
        plsc.subcore_barrier()

        def out(j, _):
            off = s * PER_SUB + j * KE
            pltpu.sync_copy(acc.at[pl.ds(off, KE)], rows)
            pltpu.sync_copy(rows, out_hbm.at[c, pl.ds(off, KE)])
            return _

        lax.fori_loop(0, PER_SUB // KE, out, None)

    return scatter_kernel


_scatter = _make_scatter()


_R = 2000


def _dinv_of(degp):
    return lax.rsqrt(degp[0] + degp[1] - 1.0)


def _t1_body(x_ref, w_ref, degp_ref, y_ref):
    dinv = _dinv_of(degp_ref[...])
    y_ref[...] = dinv * jnp.dot(
        x_ref[...], w_ref[...], preferred_element_type=jnp.float32
    )


def _t2_body(s_ref, y1_ref, degp_ref, b1_ref, w2_ref, y2_ref):
    dinv = _dinv_of(degp_ref[...])
    sblk = s_ref[...]
    h = jnp.maximum(dinv * (sblk[0] + sblk[1] + y1_ref[...]) + b1_ref[...], 0.0)
    y2_ref[...] = dinv * jnp.dot(h, w2_ref[...], preferred_element_type=jnp.float32)


def _t3_body(s_ref, y2_ref, degp_ref, b2_ref, o_ref):
    dinv = _dinv_of(degp_ref[...])
    sblk = s_ref[...]
    tot = (sblk[0] + sblk[1] + y2_ref[...])[:, :D_OUT]
    o_ref[...] = jnp.maximum(dinv * tot + b2_ref[...], 0.0)


def _row_spec(d):
    return pl.BlockSpec((_R, d), lambda i: (i, 0))


def _slab_spec(d):
    return pl.BlockSpec((NC, _R, d), lambda i: (0, i, 0))


def _full_spec(a, b):
    return pl.BlockSpec((a, b), lambda i: (0, 0))


_t1 = pl.pallas_call(
    _t1_body,
    grid=(N // _R,),
    in_specs=[_row_spec(D_IN), _full_spec(D_IN, D_HID), _slab_spec(1)],
    out_specs=_row_spec(D_HID),
    out_shape=jax.ShapeDtypeStruct((N, D_HID), jnp.float32),
)

_t2 = pl.pallas_call(
    _t2_body,
    grid=(N // _R,),
    in_specs=[
        _slab_spec(D_HID),
        _row_spec(D_HID),
        _slab_spec(1),
        _full_spec(1, D_HID),
        _full_spec(D_HID, D_HID),
    ],
    out_specs=_row_spec(D_HID),
    out_shape=jax.ShapeDtypeStruct((N, D_HID), jnp.float32),
)

_t3 = pl.pallas_call(
    _t3_body,
    grid=(N // _R,),
    in_specs=[_slab_spec(D_HID), _row_spec(D_HID), _slab_spec(1), _full_spec(1, D_OUT)],
    out_specs=_row_spec(D_OUT),
    out_shape=jax.ShapeDtypeStruct((N, D_OUT), jnp.float32),
)


_K_REAL = E // (NW * CHUNKS)


def _chunk32(v, fill):
    v3 = v.reshape(NW, CHUNKS, _K_REAL)
    return jnp.pad(
        v3, ((0, 0), (0, 1), (0, K - _K_REAL)), constant_values=fill
    )


def kernel(x, edge_index, W1, b1, W2, b2):
    ei = edge_index.astype(jnp.int32)
    src, dst = ei[0], ei[1]
    dstr = _chunk32(dst, DST_PAD)

    degp = _deg_kernel(dstr).reshape(NC, NPAD)
    degp3 = degp[:, :, None]

    W2p = jnp.pad(W2, ((0, 0), (0, D_HID - D_OUT)))

    y1 = _t1(x, W1, degp3)
    s1 = _scatter(y1, src, dst)
    y2 = _t2(s1, y1, degp3, b1[None, :], W2p)
    s2 = _scatter(y2, src, dst)
    return _t3(s2, y2, degp3, b2[None, :])
